# baseline (speedup 1.0000x reference)
import functools

import jax
import jax.numpy as jnp
from jax import lax
from jax.experimental import pallas as pl
from jax.experimental.pallas import tpu as pltpu
from jax.experimental.pallas import tpu_sc as plsc

N = 320000
D = 128
S = 512
NC = 2
NS = 16
NW = NC * NS
R = N // NW
B = 16
NB = R // B
L = 16
KD = D // L


def _phase1_body(x_hbm, batch_hbm, partial_hbm,
                 idx_v, buf0, buf1, acc, cvec, sem0, sem1):
  wid = lax.axis_index("s") * NC + lax.axis_index("c")
  r0 = wid * R
  minus_inf = jnp.full((L,), -jnp.inf, jnp.float32)

  pltpu.sync_copy(batch_hbm.at[pl.ds(r0, R)], idx_v)

  def init_body(i, _):
    for k in range(KD):
      acc[pl.ds(i * D + k * L, L)] = minus_inf
    return 0
  lax.fori_loop(0, S, init_body, 0)
  for k in range(KD):
    cvec[pl.ds(k * L, L)] = minus_inf

  pltpu.async_copy(x_hbm.at[pl.ds(r0 * D, B * D)], buf0, sem0)
  pltpu.async_copy(x_hbm.at[pl.ds((r0 + B) * D, B * D)], buf1, sem1)

  def block_rows(buf, base_r, prev):
    def group_body(g, prev):
      ids16 = idx_v[pl.ds(base_r + g * L, L)]
      last = ids16[L - 1]
      uniform = last == prev

      @pl.when(uniform)
      def _fast():
        rb = g * L * D
        for k in range(KD):
          m = buf[pl.ds(rb + k * L, L)]
          for i in range(1, L):
            m = jnp.maximum(m, buf[pl.ds(rb + i * D + k * L, L)])
          cvec[pl.ds(k * L, L)] = jnp.maximum(cvec[pl.ds(k * L, L)], m)

      @pl.when(jnp.logical_not(uniform))
      def _slow():
        sprev = prev
        for i in range(L):
          s = ids16[i]
          changed = s != sprev

          @pl.when(changed)
          def _flush(sprev=sprev):
            for k in range(KD):
              acc[pl.ds(sprev * D + k * L, L)] = cvec[pl.ds(k * L, L)]
              cvec[pl.ds(k * L, L)] = minus_inf

          rb = (g * L + i) * D
          for k in range(KD):
            cvec[pl.ds(k * L, L)] = jnp.maximum(cvec[pl.ds(k * L, L)],
                                                buf[pl.ds(rb + k * L, L)])
          sprev = s

      return last
    return lax.fori_loop(0, B // L, group_body, prev)

  def super_body(j, prev):
    for sub, (buf, sem) in enumerate(((buf0, sem0), (buf1, sem1))):
      b = 2 * j + sub
      pltpu.make_async_copy(x_hbm.at[pl.ds(0, B * D)], buf, sem).wait()
      prev = block_rows(buf, b * B, prev)
      nxt = jnp.minimum(b + 2, NB - 1)
      pltpu.async_copy(x_hbm.at[pl.ds((r0 + nxt * B) * D, B * D)], buf, sem)
    return prev

  prev = lax.fori_loop(0, NB // 2, super_body, idx_v[pl.ds(0, L)][0])

  pltpu.make_async_copy(x_hbm.at[pl.ds(0, B * D)], buf0, sem0).wait()
  prev = block_rows(buf0, (NB - 1) * B, prev)
  pltpu.make_async_copy(x_hbm.at[pl.ds(0, B * D)], buf1, sem1).wait()

  for k in range(KD):
    acc[pl.ds(prev * D + k * L, L)] = cvec[pl.ds(k * L, L)]

  pltpu.sync_copy(acc, partial_hbm.at[wid])


_phase1 = functools.partial(
    pl.kernel,
    out_type=jax.ShapeDtypeStruct((NW, S * D), jnp.float32),
    mesh=plsc.VectorSubcoreMesh(core_axis_name="c", subcore_axis_name="s"),
    scratch_types=[
        pltpu.VMEM((R,), jnp.int32),
        pltpu.VMEM((B * D,), jnp.float32),
        pltpu.VMEM((B * D,), jnp.float32),
        pltpu.VMEM((S * D,), jnp.float32),
        pltpu.VMEM((D,), jnp.float32),
        pltpu.SemaphoreType.DMA,
        pltpu.SemaphoreType.DMA,
    ],
)(_phase1_body)


def _merge_body(p_ref, o_ref):
  o_ref[...] = jnp.max(p_ref[...], axis=0)


def _phase2(partial):
  blk = (S * D) // 8
  return pl.pallas_call(
      _merge_body,
      out_shape=jax.ShapeDtypeStruct((S * D,), jnp.float32),
      grid=(8,),
      in_specs=[pl.BlockSpec((NW, blk), lambda i: (0, i))],
      out_specs=pl.BlockSpec((blk,), lambda i: (i,)),
  )(partial)


@jax.jit
def kernel(x, batch):
  partial = _phase1(jnp.reshape(x, (N * D,)), batch)
  return jnp.reshape(_phase2(partial), (S, D))


# speedup vs baseline: 1.4915x; 1.4915x over previous
import functools

import jax
import jax.numpy as jnp
from jax import lax
from jax.experimental import pallas as pl
from jax.experimental.pallas import tpu as pltpu
from jax.experimental.pallas import tpu_sc as plsc

N = 320000
D = 128
S = 512
NC = 2
NS = 16
NW = NC * NS
R = N // NW
B = 80
NB = R // B
NBUF = 4
L = 16
KD = D // L


def _phase1_body(x_hbm, batch_hbm, partial_hbm,
                 idx_v, buf0, buf1, buf2, buf3, acc, cvec,
                 sem0, sem1, sem2, sem3):
  bufs = (buf0, buf1, buf2, buf3)
  sems = (sem0, sem1, sem2, sem3)
  wid = lax.axis_index("s") * NC + lax.axis_index("c")
  r0 = wid * R
  minus_inf = jnp.full((L,), -jnp.inf, jnp.float32)

  pltpu.sync_copy(batch_hbm.at[pl.ds(r0, R)], idx_v)

  def init_body(i, _):
    for k in range(KD):
      acc[pl.ds(i * D + k * L, L)] = minus_inf
    return 0
  lax.fori_loop(0, S, init_body, 0)
  for k in range(KD):
    cvec[pl.ds(k * L, L)] = minus_inf

  for sub in range(NBUF):
    pltpu.async_copy(x_hbm.at[pl.ds((r0 + sub * B) * D, B * D)],
                     bufs[sub], sems[sub])

  def block_rows(buf, base_r, prev):
    def group_body(g, prev):
      ids16 = idx_v[pl.ds(base_r + g * L, L)]
      last = ids16[L - 1]
      uniform = last == prev

      @pl.when(uniform)
      def _fast():
        rb = g * L * D
        for k in range(KD):
          m = buf[pl.ds(rb + k * L, L)]
          for i in range(1, L):
            m = jnp.maximum(m, buf[pl.ds(rb + i * D + k * L, L)])
          cvec[pl.ds(k * L, L)] = jnp.maximum(cvec[pl.ds(k * L, L)], m)

      @pl.when(jnp.logical_not(uniform))
      def _slow():
        sprev = prev
        for i in range(L):
          s = ids16[i]
          changed = s != sprev

          @pl.when(changed)
          def _flush(sprev=sprev):
            for k in range(KD):
              acc[pl.ds(sprev * D + k * L, L)] = cvec[pl.ds(k * L, L)]
              cvec[pl.ds(k * L, L)] = minus_inf

          rb = (g * L + i) * D
          for k in range(KD):
            cvec[pl.ds(k * L, L)] = jnp.maximum(cvec[pl.ds(k * L, L)],
                                                buf[pl.ds(rb + k * L, L)])
          sprev = s

      return last
    return lax.fori_loop(0, B // L, group_body, prev)

  def super_body(j, prev):
    for sub in range(NBUF):
      buf, sem = bufs[sub], sems[sub]
      b = NBUF * j + sub
      pltpu.make_async_copy(x_hbm.at[pl.ds(0, B * D)], buf, sem).wait()
      prev = block_rows(buf, b * B, prev)
      nxt = jnp.minimum(b + NBUF, NB - 1)
      pltpu.async_copy(x_hbm.at[pl.ds((r0 + nxt * B) * D, B * D)], buf, sem)
    return prev

  prev = lax.fori_loop(0, NB // NBUF, super_body, idx_v[pl.ds(0, L)][0])

  pltpu.make_async_copy(x_hbm.at[pl.ds(0, B * D)], buf0, sem0).wait()
  prev = block_rows(buf0, (NB - 1) * B, prev)
  for sub in range(1, NBUF):
    pltpu.make_async_copy(x_hbm.at[pl.ds(0, B * D)], bufs[sub],
                          sems[sub]).wait()

  for k in range(KD):
    acc[pl.ds(prev * D + k * L, L)] = cvec[pl.ds(k * L, L)]

  pltpu.sync_copy(acc, partial_hbm.at[wid])


_phase1 = functools.partial(
    pl.kernel,
    out_type=jax.ShapeDtypeStruct((NW, S * D), jnp.float32),
    mesh=plsc.VectorSubcoreMesh(core_axis_name="c", subcore_axis_name="s"),
    scratch_types=[
        pltpu.VMEM((R,), jnp.int32),
        pltpu.VMEM((B * D,), jnp.float32),
        pltpu.VMEM((B * D,), jnp.float32),
        pltpu.VMEM((B * D,), jnp.float32),
        pltpu.VMEM((B * D,), jnp.float32),
        pltpu.VMEM((S * D,), jnp.float32),
        pltpu.VMEM((D,), jnp.float32),
        pltpu.SemaphoreType.DMA,
        pltpu.SemaphoreType.DMA,
        pltpu.SemaphoreType.DMA,
        pltpu.SemaphoreType.DMA,
    ],
)(_phase1_body)


def _merge_body(p_ref, o_ref):
  o_ref[...] = jnp.max(p_ref[...], axis=0)


def _phase2(partial):
  blk = (S * D) // 8
  return pl.pallas_call(
      _merge_body,
      out_shape=jax.ShapeDtypeStruct((S * D,), jnp.float32),
      grid=(8,),
      in_specs=[pl.BlockSpec((NW, blk), lambda i: (0, i))],
      out_specs=pl.BlockSpec((blk,), lambda i: (i,)),
  )(partial)


@jax.jit
def kernel(x, batch):
  partial = _phase1(jnp.reshape(x, (N * D,)), batch)
  return jnp.reshape(_phase2(partial), (S, D))


# speedup vs baseline: 1.6842x; 1.1292x over previous
import functools

import jax
import jax.numpy as jnp
from jax import lax
from jax.experimental import pallas as pl
from jax.experimental.pallas import tpu as pltpu
from jax.experimental.pallas import tpu_sc as plsc

N = 320000
D = 128
S = 512
NC = 2
NS = 16
NW = NC * NS
R = N // NW
B = 80
NB = R // B
NBUF = 3
L = 16
KD = D // L


def _phase1_body(x_hbm, batch_hbm, partial_hbm,
                 idx_v, buf0, buf1, buf2, acc, cvec,
                 sem0, sem1, sem2):
  bufs = (buf0, buf1, buf2)
  sems = (sem0, sem1, sem2)
  wid = lax.axis_index("s") * NC + lax.axis_index("c")
  r0 = wid * R
  minus_inf = jnp.full((L,), -jnp.inf, jnp.float32)

  pltpu.sync_copy(batch_hbm.at[pl.ds(r0, R)], idx_v)

  def init_body(i, _):
    for k in range(KD):
      acc[pl.ds(i * D + k * L, L)] = minus_inf
    return 0
  lax.fori_loop(0, S, init_body, 0)
  for k in range(KD):
    cvec[pl.ds(k * L, L)] = minus_inf

  for sub in range(2):
    pltpu.async_copy(x_hbm.at[pl.ds((r0 + sub * B) * D, B * D)],
                     bufs[sub], sems[sub])

  def block_rows(buf, base_r, prev):
    def group_body(g, prev):
      ids16 = idx_v[pl.ds(base_r + g * L, L)]
      last = ids16[L - 1]
      uniform = last == prev

      @pl.when(uniform)
      def _fast():
        rb = g * L * D
        for k in range(KD):
          m = buf[pl.ds(rb + k * L, L)]
          for i in range(1, L):
            m = jnp.maximum(m, buf[pl.ds(rb + i * D + k * L, L)])
          cvec[pl.ds(k * L, L)] = jnp.maximum(cvec[pl.ds(k * L, L)], m)

      @pl.when(jnp.logical_not(uniform))
      def _slow():
        sprev = prev
        for i in range(L):
          s = ids16[i]
          changed = s != sprev

          @pl.when(changed)
          def _flush(sprev=sprev):
            for k in range(KD):
              acc[pl.ds(sprev * D + k * L, L)] = cvec[pl.ds(k * L, L)]
              cvec[pl.ds(k * L, L)] = minus_inf

          rb = (g * L + i) * D
          for k in range(KD):
            cvec[pl.ds(k * L, L)] = jnp.maximum(cvec[pl.ds(k * L, L)],
                                                buf[pl.ds(rb + k * L, L)])
          sprev = s

      return last
    return lax.fori_loop(0, B // L, group_body, prev)

  def super_body(j, prev):
    for sub in range(NBUF):
      buf, sem = bufs[sub], sems[sub]
      b = NBUF * j + sub
      pltpu.make_async_copy(x_hbm.at[pl.ds(0, B * D)], buf, sem).wait()
      pltpu.async_copy(x_hbm.at[pl.ds((r0 + (b + 2) * B) * D, B * D)],
                       bufs[(sub + 2) % NBUF], sems[(sub + 2) % NBUF])
      prev = block_rows(buf, b * B, prev)
    return prev

  prev = lax.fori_loop(0, NB // NBUF, super_body, idx_v[pl.ds(0, L)][0])

  pltpu.make_async_copy(x_hbm.at[pl.ds(0, B * D)], buf0, sem0).wait()
  prev = block_rows(buf0, (NB - 2) * B, prev)
  pltpu.make_async_copy(x_hbm.at[pl.ds(0, B * D)], buf1, sem1).wait()
  prev = block_rows(buf1, (NB - 1) * B, prev)

  for k in range(KD):
    acc[pl.ds(prev * D + k * L, L)] = cvec[pl.ds(k * L, L)]

  pltpu.sync_copy(acc, partial_hbm.at[wid])


_phase1 = functools.partial(
    pl.kernel,
    out_type=jax.ShapeDtypeStruct((NW, S * D), jnp.float32),
    mesh=plsc.VectorSubcoreMesh(core_axis_name="c", subcore_axis_name="s"),
    scratch_types=[
        pltpu.VMEM((R,), jnp.int32),
        pltpu.VMEM((B * D,), jnp.float32),
        pltpu.VMEM((B * D,), jnp.float32),
        pltpu.VMEM((B * D,), jnp.float32),
        pltpu.VMEM((S * D,), jnp.float32),
        pltpu.VMEM((D,), jnp.float32),
        pltpu.SemaphoreType.DMA,
        pltpu.SemaphoreType.DMA,
        pltpu.SemaphoreType.DMA,
    ],
)(_phase1_body)


def _merge_body(p_ref, o_ref):
  o_ref[...] = jnp.max(p_ref[...], axis=0)


def _phase2(partial):
  blk = (S * D) // 8
  return pl.pallas_call(
      _merge_body,
      out_shape=jax.ShapeDtypeStruct((S * D,), jnp.float32),
      grid=(8,),
      in_specs=[pl.BlockSpec((NW, blk), lambda i: (0, i))],
      out_specs=pl.BlockSpec((blk,), lambda i: (i,)),
  )(partial)


@jax.jit
def kernel(x, batch):
  partial = _phase1(jnp.reshape(x, (N * D,)), batch)
  return jnp.reshape(_phase2(partial), (S, D))


# speedup vs baseline: 2.0674x; 1.2275x over previous
import functools

import jax
import jax.numpy as jnp
from jax import lax
from jax.experimental import pallas as pl
from jax.experimental.pallas import tpu as pltpu
from jax.experimental.pallas import tpu_sc as plsc

N = 320000
D = 128
S = 512
NC = 2
NS = 16
NW = NC * NS
R = N // NW
B = 400
NB = R // B
L = 16
KD = D // L
DM = 128
GR = 8


def _phase1_body(x_hbm, batch_hbm, partial_hbm, meta_hbm,
                 idx_v, buf0, buf1, cvec, ivec, mvec, sem0, sem1):
  wid = lax.axis_index("s") * NC + lax.axis_index("c")
  r0 = wid * R
  pbase = wid * S * D
  minus_inf = jnp.full((L,), -jnp.inf, jnp.float32)

  pltpu.sync_copy(batch_hbm.at[pl.ds(r0, R)], idx_v)

  for k in range(KD):
    cvec[pl.ds(k * L, L)] = minus_inf
    ivec[pl.ds(k * L, L)] = minus_inf

  pltpu.async_copy(x_hbm.at[pl.ds(r0 * D, B * D)], buf0, sem0)
  pltpu.async_copy(x_hbm.at[pl.ds((r0 + B) * D, B * D)], buf1, sem1)

  def block_rows(buf, base_r, prev):
    def group_body(g, prev):
      ids16 = idx_v[pl.ds(base_r + g * L, L)]
      last = ids16[L - 1]
      uniform = last == prev

      @pl.when(uniform)
      def _fast():
        rb = g * L * D
        for k in range(KD):
          m = buf[pl.ds(rb + k * L, L)]
          for i in range(1, L):
            m = jnp.maximum(m, buf[pl.ds(rb + i * D + k * L, L)])
          cvec[pl.ds(k * L, L)] = jnp.maximum(cvec[pl.ds(k * L, L)], m)

      @pl.when(jnp.logical_not(uniform))
      def _slow():
        sprev = prev
        for i in range(L):
          s = ids16[i]
          changed = s != sprev

          @pl.when(changed)
          def _flush(sprev=sprev, s=s):
            pltpu.sync_copy(cvec,
                            partial_hbm.at[pl.ds(pbase + sprev * D, D)])
            for k in range(KD):
              cvec[pl.ds(k * L, L)] = minus_inf

            def gap_body(gseg, _):
              pltpu.sync_copy(ivec,
                              partial_hbm.at[pl.ds(pbase + gseg * D, D)])
              return 0
            lax.fori_loop(sprev + 1, s, gap_body, 0)

          rb = (g * L + i) * D
          for k in range(KD):
            cvec[pl.ds(k * L, L)] = jnp.maximum(cvec[pl.ds(k * L, L)],
                                                buf[pl.ds(rb + k * L, L)])
          sprev = s

      return last
    return lax.fori_loop(0, B // L, group_body, prev)

  lo = idx_v[pl.ds(0, L)][0]

  def super_body(j, prev):
    for sub, (buf, sem) in enumerate(((buf0, sem0), (buf1, sem1))):
      b = 2 * j + sub
      pltpu.make_async_copy(x_hbm.at[pl.ds(0, B * D)], buf, sem).wait()
      prev = block_rows(buf, b * B, prev)
      nxt = jnp.minimum(b + 2, NB - 1)
      pltpu.async_copy(x_hbm.at[pl.ds((r0 + nxt * B) * D, B * D)], buf, sem)
    return prev

  prev = lax.fori_loop(0, NB // 2, super_body, lo)

  pltpu.make_async_copy(x_hbm.at[pl.ds(0, B * D)], buf0, sem0).wait()
  prev = block_rows(buf0, (NB - 1) * B, prev)
  pltpu.make_async_copy(x_hbm.at[pl.ds(0, B * D)], buf1, sem1).wait()

  pltpu.sync_copy(cvec, partial_hbm.at[pl.ds(pbase + prev * D, D)])
  lane = lax.iota(jnp.int32, L)
  mvec[pl.ds(0, L)] = jnp.where(lane == 0, lo, prev)
  pltpu.sync_copy(mvec, meta_hbm.at[pl.ds(wid * DM, L)])


_phase1 = functools.partial(
    pl.kernel,
    out_type=(jax.ShapeDtypeStruct((NW * S * D,), jnp.float32),
              jax.ShapeDtypeStruct((NW * DM,), jnp.int32)),
    mesh=plsc.VectorSubcoreMesh(core_axis_name="c", subcore_axis_name="s"),
    scratch_types=[
        pltpu.VMEM((R,), jnp.int32),
        pltpu.VMEM((B * D,), jnp.float32),
        pltpu.VMEM((B * D,), jnp.float32),
        pltpu.VMEM((D,), jnp.float32),
        pltpu.VMEM((D,), jnp.float32),
        pltpu.VMEM((L,), jnp.int32),
        pltpu.SemaphoreType.DMA,
        pltpu.SemaphoreType.DMA,
    ],
)(_phase1_body)


def _merge_body(m_ref, p_ref, o_ref):
  i = pl.program_id(0)
  blk = (S * D) // GR
  m = m_ref[...]
  lo = m[:, 0:1]
  hi = m[:, 1:2]
  seg = i * (blk // D) + lax.broadcasted_iota(jnp.int32, (1, blk), 1) // D
  mask = (lo <= seg) & (seg <= hi)
  o_ref[...] = jnp.max(jnp.where(mask, p_ref[...], -jnp.inf), axis=0)


def _phase2(partial, meta):
  blk = (S * D) // GR
  return pl.pallas_call(
      _merge_body,
      out_shape=jax.ShapeDtypeStruct((S * D,), jnp.float32),
      grid=(GR,),
      in_specs=[pl.BlockSpec((NW, DM), lambda i: (0, 0)),
                pl.BlockSpec((NW, blk), lambda i: (0, i))],
      out_specs=pl.BlockSpec((blk,), lambda i: (i,)),
  )(meta, partial)


@jax.jit
def kernel(x, batch):
  partial, meta = _phase1(jnp.reshape(x, (N * D,)), batch)
  out = _phase2(jnp.reshape(partial, (NW, S * D)),
                jnp.reshape(meta, (NW, DM)))
  return jnp.reshape(out, (S, D))
